# baseline (speedup 1.0000x reference)
import functools
import math

import jax
import jax.numpy as jnp
from jax import lax
from jax.experimental import pallas as pl
from jax.experimental.pallas import tpu as pltpu
from jax.experimental.pallas import tpu_sc as plsc

VOCAB = 1000000
D = 64
ROWS = 4096
COLS = 200
B_TOT = ROWS * COLS
NC = 2
NS = 16
NW = NC * NS
B_PER_W = B_TOT // NW
CHUNK = 128
N_CHUNKS = B_PER_W // CHUNK
LANES = 16
SCALE = math.sqrt(D)

_mesh = plsc.VectorSubcoreMesh(core_axis_name="c", subcore_axis_name="s")


@functools.partial(
    pl.kernel,
    mesh=_mesh,
    out_type=jax.ShapeDtypeStruct((NW, N_CHUNKS, CHUNK, D), jnp.float32),
    compiler_params=pltpu.CompilerParams(use_tc_tiling_on_sc=False),
    scratch_types=[
        pltpu.VMEM((N_CHUNKS, CHUNK), jnp.int32),
        pltpu.VMEM((CHUNK, D), jnp.float32),
        pltpu.SemaphoreType.DMA,
    ],
)
def _emb_lookup(x_hbm, table_hbm, out_hbm, idx_v, rows_v, gsem):
    wid = lax.axis_index("s") * NC + lax.axis_index("c")
    pltpu.sync_copy(x_hbm.at[wid], idx_v)

    def chunk_body(j, carry):
        pltpu.async_copy(table_hbm.at[idx_v.at[j]], rows_v, gsem).wait()

        def row_body(i, c):
            for t in range(D // LANES):
                sl = pl.ds(t * LANES, LANES)
                rows_v[i, sl] = rows_v[i, sl] * SCALE
            return c

        lax.fori_loop(0, CHUNK, row_body, 0, unroll=4)
        pltpu.sync_copy(rows_v, out_hbm.at[wid, j])
        return carry

    lax.fori_loop(0, N_CHUNKS, chunk_body, 0)


def kernel(x, table):
    idx = x.astype(jnp.int32).reshape(NW, N_CHUNKS, CHUNK)
    out = _emb_lookup(idx, table)
    return out.reshape(ROWS, COLS, D)


# speedup vs baseline: 1.0580x; 1.0580x over previous
import functools
import math

import jax
import jax.numpy as jnp
from jax import lax
from jax.experimental import pallas as pl
from jax.experimental.pallas import tpu as pltpu
from jax.experimental.pallas import tpu_sc as plsc

VOCAB = 1000000
D = 64
ROWS = 4096
COLS = 200
B_TOT = ROWS * COLS
NC = 2
NS = 16
NW = NC * NS
B_PER_W = B_TOT // NW
CHUNK = 128
N_CHUNKS = B_PER_W // CHUNK
NBUF = 4
LANES = 16
SCALE = math.sqrt(D)

_mesh = plsc.VectorSubcoreMesh(core_axis_name="c", subcore_axis_name="s")


@functools.partial(
    pl.kernel,
    mesh=_mesh,
    out_type=jax.ShapeDtypeStruct((NW, N_CHUNKS, CHUNK, D), jnp.float32),
    compiler_params=pltpu.CompilerParams(use_tc_tiling_on_sc=False),
    scratch_types=[
        pltpu.VMEM((N_CHUNKS, CHUNK), jnp.int32),
        pltpu.VMEM((NBUF, CHUNK, D), jnp.float32),
        pltpu.VMEM((NBUF, CHUNK, D), jnp.float32),
    ]
    + [pltpu.SemaphoreType.DMA] * (2 * NBUF),
)
def _emb_lookup(x_hbm, table_hbm, out_hbm, idx_v, in_v, out_v, *sems):
    gsem = sems[:NBUF]
    ssem = sems[NBUF:]
    wid = lax.axis_index("s") * NC + lax.axis_index("c")
    pltpu.sync_copy(x_hbm.at[wid], idx_v)

    for b in range(NBUF):
        pltpu.async_copy(table_hbm.at[idx_v.at[b]], in_v.at[b], gsem[b])

    def group_body(g, carry):
        for b in range(NBUF):
            jj = g * NBUF + b
            pltpu.make_async_copy(
                table_hbm.at[idx_v.at[b]], in_v.at[b], gsem[b]
            ).wait()

            @pl.when(jj >= NBUF)
            def _():
                pltpu.make_async_copy(
                    out_v.at[b], out_hbm.at[wid, 0], ssem[b]
                ).wait()

            def row_body(i, c):
                for t in range(D // LANES):
                    sl = pl.ds(t * LANES, LANES)
                    out_v[b, i, sl] = in_v[b, i, sl] * SCALE
                return c

            lax.fori_loop(0, CHUNK, row_body, 0, unroll=8)
            pltpu.async_copy(out_v.at[b], out_hbm.at[wid, jj], ssem[b])

            @pl.when(jj + NBUF < N_CHUNKS)
            def _():
                pltpu.async_copy(
                    table_hbm.at[idx_v.at[jj + NBUF]], in_v.at[b], gsem[b]
                )

        return carry

    lax.fori_loop(0, N_CHUNKS // NBUF, group_body, 0)
    for b in range(NBUF):
        pltpu.make_async_copy(out_v.at[b], out_hbm.at[wid, 0], ssem[b]).wait()


def kernel(x, table):
    idx = x.astype(jnp.int32).reshape(NW, N_CHUNKS, CHUNK)
    out = _emb_lookup(idx, table)
    return out.reshape(ROWS, COLS, D)
